# baseline (speedup 1.0000x reference)
import functools

import jax
import jax.numpy as jnp
from jax import lax
from jax.experimental import pallas as pl
from jax.experimental.pallas import tpu as pltpu
from jax.experimental.pallas import tpu_sc as plsc

N = 10000
E = 160000
D = 128
HID = 128
NH = 4
HC = 32
NC = 8
F_M = NH * HC * NC
FT = F_M + HID
CW = 128
NCH = FT // CW

NSC = 2
NTILE = 16
EPT = E // NTILE
EB = 125
NB = EPT // EB
NP = 10240
RPT = NP // NTILE

BN = 1000
GRID = N // BN

_HIGH = lax.Precision.HIGHEST


def _sel_matrices():
    cc = lax.broadcasted_iota(jnp.int32, (HID, F_M), 1)
    rr = lax.broadcasted_iota(jnp.int32, (HID, F_M), 0)
    sk = (rr == (cc // (HC * NC)) * HC + (cc % (HC * NC)) // NC)
    return sk.astype(jnp.float32)


def _elu1(a):
    return 1.0 + jnp.where(a > 0, a, jnp.exp(jnp.minimum(a, 0.0)) - 1.0)


def _prep_body(x_ref, win_ref, bin_ref, wq_ref, bq_ref, wk_ref, bk_ref,
               wv_ref, bv_ref, q_ref, v_ref, *m0_refs):
    xb = x_ref[...]
    h = jnp.maximum(jnp.dot(xb, win_ref[...], precision=_HIGH) + bin_ref[...], 0.0)
    q = _elu1(jnp.dot(h, wq_ref[...], precision=_HIGH) + bq_ref[...])
    km = _elu1(jnp.dot(h, wk_ref[...], precision=_HIGH) + bk_ref[...])
    v = jnp.dot(h, wv_ref[...], precision=_HIGH) + bv_ref[...]
    q_ref[...] = q
    v_ref[...] = v
    sk = _sel_matrices()
    cc2 = lax.broadcasted_iota(jnp.int32, (NH * NC, F_M), 1)
    rr2 = lax.broadcasted_iota(jnp.int32, (NH * NC, F_M), 0)
    sv = (rr2 == (cc2 // (HC * NC)) * NC + cc2 % NC).astype(jnp.float32)
    m0 = jnp.dot(km, sk, precision=_HIGH) * jnp.dot(v, sv, precision=_HIGH)
    feat = jnp.concatenate([m0, km], axis=1)
    for c in range(NCH):
        m0_refs[c][...] = feat[:, c * CW:(c + 1) * CW]


def _tc_prep(x, w_in, b_in, wq, bq, wk, bk, wv, bv):
    node_spec = lambda w: pl.BlockSpec((BN, w), lambda i: (i, 0))
    full_spec = lambda a: pl.BlockSpec(a.shape, lambda i: (0,) * a.ndim)
    out_shape = ([jax.ShapeDtypeStruct((N, HID), jnp.float32),
                  jax.ShapeDtypeStruct((N, NH * NC), jnp.float32)] +
                 [jax.ShapeDtypeStruct((N, CW), jnp.float32) for _ in range(NCH)])
    out_specs = ([node_spec(HID), node_spec(NH * NC)] +
                 [node_spec(CW) for _ in range(NCH)])
    ws = (w_in, b_in, wq, bq, wk, bk, wv, bv)
    return pl.pallas_call(
        _prep_body,
        grid=(GRID,),
        in_specs=[node_spec(D)] + [full_spec(a) for a in ws],
        out_specs=out_specs,
        out_shape=out_shape,
    )(x, *ws)


def _sc_segsum(row2d, col2d, xs):
    nx = len(xs)
    mesh = plsc.VectorSubcoreMesh(core_axis_name="c", subcore_axis_name="s",
                                  num_cores=NSC, num_subcores=NTILE)
    out_type = [jax.ShapeDtypeStruct((NP, CW), jnp.float32) for _ in range(nx + 1)]
    scratch = [
        pltpu.VMEM((8, EB), jnp.int32),
        pltpu.VMEM((8, EB), jnp.int32),
        pltpu.VMEM((EB, CW), jnp.float32),
        pltpu.VMEM((EB, CW), jnp.float32),
        pltpu.VMEM((112, CW), jnp.float32),
        pltpu.VMEM_SHARED((NP, CW), jnp.float32),
        pltpu.SemaphoreType.DMA,
        pltpu.SemaphoreType.DMA,
        pltpu.SemaphoreType.DMA,
        pltpu.SemaphoreType.DMA,
    ]

    def body(row_ref, col_ref, *rest):
        x_refs = rest[:nx]
        y_refs = rest[nx:2 * nx + 1]
        ridx, cidx, rows0, rows1, zbuf, acc, g0, g1, s0, s1 = rest[2 * nx + 1:]
        rows = (rows0, rows1)
        gsem = (g0, g1)
        ssem = (s0, s1)
        cid = lax.axis_index("c")
        sid = lax.axis_index("s")

        def zloop(i, carry):
            for j in range(CW // 16):
                zbuf[i, pl.ds(j * 16, 16)] = jnp.zeros((16,), jnp.float32)
            return carry
        lax.fori_loop(0, 112, zloop, 0)

        def process(x_ref, y_ref, ngroups, goff):
            for z in range(5):
                pltpu.sync_copy(zbuf, acc.at[pl.ds(sid * RPT + z * 112, 112)])
            pltpu.sync_copy(zbuf.at[pl.ds(0, 80)],
                            acc.at[pl.ds(sid * RPT + 560, 80)])
            plsc.subcore_barrier()

            def oloop(bb, carry):
                base = pl.multiple_of(sid * NB + goff + bb * 8, 8)
                pltpu.sync_copy(row_ref.at[pl.ds(base, 8)], ridx)
                pltpu.sync_copy(col_ref.at[pl.ds(base, 8)], cidx)
                pltpu.async_copy(x_ref.at[ridx.at[0]], rows[0], gsem[0])
                for k in range(8):
                    cur, nxt = k % 2, (k + 1) % 2
                    pltpu.make_async_copy(
                        x_ref.at[ridx.at[k]], rows[cur], gsem[cur]).wait()
                    if k >= 1:
                        pltpu.make_async_copy(
                            rows[nxt], acc.at[cidx.at[k - 1]], ssem[nxt]).wait()
                    if k < 7:
                        pltpu.async_copy(
                            x_ref.at[ridx.at[k + 1]], rows[nxt], gsem[nxt])
                    pltpu.async_copy(rows[cur], acc.at[cidx.at[k]],
                                     ssem[cur], add=True)
                pltpu.make_async_copy(rows[1], acc.at[cidx.at[7]], ssem[1]).wait()
                return carry
            lax.fori_loop(0, ngroups, oloop, 0)
            plsc.subcore_barrier()
            pltpu.sync_copy(acc.at[pl.ds(sid * RPT, RPT)],
                            y_ref.at[pl.ds(sid * RPT, RPT)])

        for c in range(nx - 1):
            @pl.when(cid == (c % NSC))
            def _(c=c):
                process(x_refs[c], y_refs[c], NB // 8, 0)

        @pl.when(cid == 0)
        def _():
            process(x_refs[nx - 1], y_refs[nx - 1], NB // 16, 0)

        @pl.when(cid == 1)
        def _():
            process(x_refs[nx - 1], y_refs[nx], NB // 16, NB // 2)

    fn = pl.kernel(body, out_type=out_type, mesh=mesh, scratch_types=scratch)
    return fn(row2d, col2d, *xs)


def _add_body(a_ref, b_ref, o_ref):
    o_ref[...] = a_ref[...] + b_ref[...]


def _tc_add(a, b):
    spec = pl.BlockSpec((NP // 8, CW), lambda i: (i, 0))
    return pl.pallas_call(
        _add_body, grid=(8,), in_specs=[spec, spec], out_specs=spec,
        out_shape=jax.ShapeDtypeStruct((NP, CW), jnp.float32))(a, b)


def _final_body(q_ref, v_ref, hwe_ref, wout_ref, bout_ref, *rest):
    f1_refs = rest[:NCH]
    f2_refs = rest[NCH:2 * NCH + 1]
    out_ref = rest[2 * NCH + 1]
    qb = q_ref[...]
    sk = _sel_matrices()
    qexp = jnp.dot(qb, sk, precision=_HIGH)
    ccH = lax.broadcasted_iota(jnp.int32, (F_M, NH * NC), 0)
    ocH = lax.broadcasted_iota(jnp.int32, (F_M, NH * NC), 1)
    rh = (ocH == (ccH // (HC * NC)) * NC + ccH % NC).astype(jnp.float32)
    rrC = lax.broadcasted_iota(jnp.int32, (HID, NH * NC), 0)
    ocC = lax.broadcasted_iota(jnp.int32, (HID, NH * NC), 1)
    rc = (ocC // NC == rrC // HC).astype(jnp.float32)

    f1 = jnp.concatenate([r[...] for r in f1_refs], axis=1)
    f2 = jnp.concatenate([r[...] for r in f2_refs[:NCH - 1]] +
                         [f2_refs[NCH - 1][...] + f2_refs[NCH][...]], axis=1)
    m1, km1 = f1[:, :F_M], f1[:, F_M:]
    m2, km2 = f2[:, :F_M], f2[:, F_M:]
    h1 = jnp.dot(qexp * m1, rh, precision=_HIGH)
    c1 = jnp.dot(qb * km1, rc, precision=_HIGH)
    h2 = jnp.dot(qexp * m2, rh, precision=_HIGH)
    c2 = jnp.dot(qb * km2, rc, precision=_HIGH)
    hw = hwe_ref[...]
    hid = (v_ref[...] * hw[0:1, :] + hw[1:2, :] * h1 / (c1 + 1e-5)
           + hw[2:3, :] * h2 / (c2 + 1e-5))
    out_ref[...] = jnp.dot(hid, wout_ref[...], precision=_HIGH) + bout_ref[...]


def _tc_final(q, v, hwe, w_out, b_out, f1s, f2s):
    node_spec = lambda w: pl.BlockSpec((BN, w), lambda i: (i, 0))
    full_spec = lambda a: pl.BlockSpec(a.shape, lambda i: (0,) * a.ndim)
    ins = [q, v]
    in_specs = [node_spec(HID), node_spec(NH * NC)]
    ins += [hwe, w_out, b_out]
    in_specs += [full_spec(hwe), full_spec(w_out), full_spec(b_out)]
    ins += list(f1s) + list(f2s)
    in_specs += [node_spec(CW)] * (len(f1s) + len(f2s))
    return pl.pallas_call(
        _final_body,
        grid=(GRID,),
        in_specs=in_specs,
        out_specs=node_spec(NC),
        out_shape=jax.ShapeDtypeStruct((N, NC), jnp.float32),
    )(*ins)


def kernel(x, edge_index, W_in, b_in, WQ, bQ, WK, bK, WV, bV, W_out, b_out, hopwise):
    row2d = edge_index[0].reshape(NTILE * NB, EB)
    col2d = edge_index[1].reshape(NTILE * NB, EB)
    b_in2 = b_in.reshape(1, HID)
    bq2 = bQ.reshape(1, HID)
    bk2 = bK.reshape(1, HID)
    bv2 = bV.reshape(1, NH * NC)
    bout2 = b_out.reshape(1, NC)
    hwe = jnp.repeat(hopwise.T, NC, axis=1)

    q, v, *f0s = _tc_prep(x, W_in, b_in2, WQ, bq2, WK, bk2, WV, bv2)
    ys1 = _sc_segsum(row2d, col2d, list(f0s))
    km1 = _tc_add(ys1[NCH - 1], ys1[NCH])
    f1s = ys1[:NCH - 1] + [km1]
    ys2 = _sc_segsum(row2d, col2d, f1s)
    return _tc_final(q, v, hwe, W_out, bout2, f1s, ys2)


# speedup vs baseline: 1.0146x; 1.0146x over previous
import functools

import jax
import jax.numpy as jnp
from jax import lax
from jax.experimental import pallas as pl
from jax.experimental.pallas import tpu as pltpu
from jax.experimental.pallas import tpu_sc as plsc

N = 10000
E = 160000
D = 128
HID = 128
NH = 4
HC = 32
NC = 8
F_M = NH * HC * NC
FT = F_M + HID
CW = 128
NCH = FT // CW

NSC = 2
NTILE = 16
EPT = E // NTILE
EB = 125
NB = EPT // EB
NP = 10240
RPT = NP // NTILE

BN = 400
GRID = N // BN

_HIGH = lax.Precision.HIGHEST


def _sel_matrices():
    cc = lax.broadcasted_iota(jnp.int32, (HID, F_M), 1)
    rr = lax.broadcasted_iota(jnp.int32, (HID, F_M), 0)
    sk = (rr == (cc // (HC * NC)) * HC + (cc % (HC * NC)) // NC)
    return sk.astype(jnp.float32)


def _elu1(a):
    return 1.0 + jnp.where(a > 0, a, jnp.exp(jnp.minimum(a, 0.0)) - 1.0)


def _prep_body(x_ref, win_ref, bin_ref, wq_ref, bq_ref, wk_ref, bk_ref,
               wv_ref, bv_ref, q_ref, v_ref, *m0_refs):
    xb = x_ref[...]
    h = jnp.maximum(jnp.dot(xb, win_ref[...], precision=_HIGH) + bin_ref[...], 0.0)
    q = _elu1(jnp.dot(h, wq_ref[...], precision=_HIGH) + bq_ref[...])
    km = _elu1(jnp.dot(h, wk_ref[...], precision=_HIGH) + bk_ref[...])
    v = jnp.dot(h, wv_ref[...], precision=_HIGH) + bv_ref[...]
    q_ref[...] = q
    v_ref[...] = v
    sk = _sel_matrices()
    cc2 = lax.broadcasted_iota(jnp.int32, (NH * NC, F_M), 1)
    rr2 = lax.broadcasted_iota(jnp.int32, (NH * NC, F_M), 0)
    sv = (rr2 == (cc2 // (HC * NC)) * NC + cc2 % NC).astype(jnp.float32)
    m0 = jnp.dot(km, sk, precision=_HIGH) * jnp.dot(v, sv, precision=_HIGH)
    feat = jnp.concatenate([m0, km], axis=1)
    for c in range(NCH):
        m0_refs[c][...] = feat[:, c * CW:(c + 1) * CW]


def _tc_prep(x, w_in, b_in, wq, bq, wk, bk, wv, bv):
    node_spec = lambda w: pl.BlockSpec((BN, w), lambda i: (i, 0))
    full_spec = lambda a: pl.BlockSpec(a.shape, lambda i: (0,) * a.ndim)
    out_shape = ([jax.ShapeDtypeStruct((N, HID), jnp.float32),
                  jax.ShapeDtypeStruct((N, NH * NC), jnp.float32)] +
                 [jax.ShapeDtypeStruct((N, CW), jnp.float32) for _ in range(NCH)])
    out_specs = ([node_spec(HID), node_spec(NH * NC)] +
                 [node_spec(CW) for _ in range(NCH)])
    ws = (w_in, b_in, wq, bq, wk, bk, wv, bv)
    return pl.pallas_call(
        _prep_body,
        grid=(GRID,),
        in_specs=[node_spec(D)] + [full_spec(a) for a in ws],
        out_specs=out_specs,
        out_shape=out_shape,
    )(x, *ws)


def _sc_segsum(row2d, col2d, xs):
    nx = len(xs)
    mesh = plsc.VectorSubcoreMesh(core_axis_name="c", subcore_axis_name="s",
                                  num_cores=NSC, num_subcores=NTILE)
    out_type = [jax.ShapeDtypeStruct((NP, CW), jnp.float32) for _ in range(nx + 1)]
    scratch = [
        pltpu.VMEM((8, EB), jnp.int32),
        pltpu.VMEM((8, EB), jnp.int32),
        pltpu.VMEM((EB, CW), jnp.float32),
        pltpu.VMEM((EB, CW), jnp.float32),
        pltpu.VMEM((112, CW), jnp.float32),
        pltpu.VMEM_SHARED((NP, CW), jnp.float32),
        pltpu.SemaphoreType.DMA,
        pltpu.SemaphoreType.DMA,
        pltpu.SemaphoreType.DMA,
        pltpu.SemaphoreType.DMA,
    ]

    def body(row_ref, col_ref, *rest):
        x_refs = rest[:nx]
        y_refs = rest[nx:2 * nx + 1]
        ridx, cidx, rows0, rows1, zbuf, acc, g0, g1, s0, s1 = rest[2 * nx + 1:]
        rows = (rows0, rows1)
        gsem = (g0, g1)
        ssem = (s0, s1)
        cid = lax.axis_index("c")
        sid = lax.axis_index("s")

        def zloop(i, carry):
            for j in range(CW // 16):
                zbuf[i, pl.ds(j * 16, 16)] = jnp.zeros((16,), jnp.float32)
            return carry
        lax.fori_loop(0, 112, zloop, 0)

        def process(x_ref, y_ref, ngroups, goff):
            for z in range(5):
                pltpu.sync_copy(zbuf, acc.at[pl.ds(sid * RPT + z * 112, 112)])
            pltpu.sync_copy(zbuf.at[pl.ds(0, 80)],
                            acc.at[pl.ds(sid * RPT + 560, 80)])
            plsc.subcore_barrier()

            def oloop(bb, carry):
                base = pl.multiple_of(sid * NB + goff + bb * 8, 8)
                pltpu.sync_copy(row_ref.at[pl.ds(base, 8)], ridx)
                pltpu.sync_copy(col_ref.at[pl.ds(base, 8)], cidx)
                pltpu.async_copy(x_ref.at[ridx.at[0]], rows[0], gsem[0])
                for k in range(8):
                    cur, nxt = k % 2, (k + 1) % 2
                    pltpu.make_async_copy(
                        x_ref.at[ridx.at[k]], rows[cur], gsem[cur]).wait()
                    if k >= 1:
                        pltpu.make_async_copy(
                            rows[nxt], acc.at[cidx.at[k - 1]], ssem[nxt]).wait()
                    if k < 7:
                        pltpu.async_copy(
                            x_ref.at[ridx.at[k + 1]], rows[nxt], gsem[nxt])
                    pltpu.async_copy(rows[cur], acc.at[cidx.at[k]],
                                     ssem[cur], add=True)
                pltpu.make_async_copy(rows[1], acc.at[cidx.at[7]], ssem[1]).wait()
                return carry
            lax.fori_loop(0, ngroups, oloop, 0)
            plsc.subcore_barrier()
            pltpu.sync_copy(acc.at[pl.ds(sid * RPT, RPT)],
                            y_ref.at[pl.ds(sid * RPT, RPT)])

        for c in range(nx - 1):
            @pl.when(cid == (c % NSC))
            def _(c=c):
                process(x_refs[c], y_refs[c], NB // 8, 0)

        @pl.when(cid == 0)
        def _():
            process(x_refs[nx - 1], y_refs[nx - 1], NB // 16, 0)

        @pl.when(cid == 1)
        def _():
            process(x_refs[nx - 1], y_refs[nx], NB // 16, NB // 2)

    fn = pl.kernel(body, out_type=out_type, mesh=mesh, scratch_types=scratch)
    return fn(row2d, col2d, *xs)


def _add_body(a_ref, b_ref, o_ref):
    o_ref[...] = a_ref[...] + b_ref[...]


def _tc_add(a, b):
    spec = pl.BlockSpec((NP // 8, CW), lambda i: (i, 0))
    return pl.pallas_call(
        _add_body, grid=(8,), in_specs=[spec, spec], out_specs=spec,
        out_shape=jax.ShapeDtypeStruct((NP, CW), jnp.float32))(a, b)


def _final_body(q_ref, v_ref, hwe_ref, wout_ref, bout_ref, *rest):
    f1_refs = rest[:NCH]
    f2_refs = rest[NCH:2 * NCH + 1]
    out_ref = rest[2 * NCH + 1]
    qb = q_ref[...]
    sk = _sel_matrices()
    qexp = jnp.dot(qb, sk, precision=_HIGH)
    ccH = lax.broadcasted_iota(jnp.int32, (F_M, NH * NC), 0)
    ocH = lax.broadcasted_iota(jnp.int32, (F_M, NH * NC), 1)
    rh = (ocH == (ccH // (HC * NC)) * NC + ccH % NC).astype(jnp.float32)
    rrC = lax.broadcasted_iota(jnp.int32, (HID, NH * NC), 0)
    ocC = lax.broadcasted_iota(jnp.int32, (HID, NH * NC), 1)
    rc = (ocC // NC == rrC // HC).astype(jnp.float32)

    f1 = jnp.concatenate([r[...] for r in f1_refs], axis=1)
    f2 = jnp.concatenate([r[...] for r in f2_refs[:NCH - 1]] +
                         [f2_refs[NCH - 1][...] + f2_refs[NCH][...]], axis=1)
    m1, km1 = f1[:, :F_M], f1[:, F_M:]
    m2, km2 = f2[:, :F_M], f2[:, F_M:]
    h1 = jnp.dot(qexp * m1, rh, precision=_HIGH)
    c1 = jnp.dot(qb * km1, rc, precision=_HIGH)
    h2 = jnp.dot(qexp * m2, rh, precision=_HIGH)
    c2 = jnp.dot(qb * km2, rc, precision=_HIGH)
    hw = hwe_ref[...]
    hid = (v_ref[...] * hw[0:1, :] + hw[1:2, :] * h1 / (c1 + 1e-5)
           + hw[2:3, :] * h2 / (c2 + 1e-5))
    out_ref[...] = jnp.dot(hid, wout_ref[...], precision=_HIGH) + bout_ref[...]


def _tc_final(q, v, hwe, w_out, b_out, f1s, f2s):
    node_spec = lambda w: pl.BlockSpec((BN, w), lambda i: (i, 0))
    full_spec = lambda a: pl.BlockSpec(a.shape, lambda i: (0,) * a.ndim)
    ins = [q, v]
    in_specs = [node_spec(HID), node_spec(NH * NC)]
    ins += [hwe, w_out, b_out]
    in_specs += [full_spec(hwe), full_spec(w_out), full_spec(b_out)]
    ins += list(f1s) + list(f2s)
    in_specs += [node_spec(CW)] * (len(f1s) + len(f2s))
    return pl.pallas_call(
        _final_body,
        grid=(GRID,),
        in_specs=in_specs,
        out_specs=node_spec(NC),
        out_shape=jax.ShapeDtypeStruct((N, NC), jnp.float32),
    )(*ins)


def kernel(x, edge_index, W_in, b_in, WQ, bQ, WK, bK, WV, bV, W_out, b_out, hopwise):
    row2d = edge_index[0].reshape(NTILE * NB, EB)
    col2d = edge_index[1].reshape(NTILE * NB, EB)
    b_in2 = b_in.reshape(1, HID)
    bq2 = bQ.reshape(1, HID)
    bk2 = bK.reshape(1, HID)
    bv2 = bV.reshape(1, NH * NC)
    bout2 = b_out.reshape(1, NC)
    hwe = jnp.repeat(hopwise.T, NC, axis=1)

    q, v, *f0s = _tc_prep(x, W_in, b_in2, WQ, bq2, WK, bk2, WV, bv2)
    ys1 = _sc_segsum(row2d, col2d, list(f0s))
    km1 = _tc_add(ys1[NCH - 1], ys1[NCH])
    f1s = ys1[:NCH - 1] + [km1]
    ys2 = _sc_segsum(row2d, col2d, f1s)
    return _tc_final(q, v, hwe, W_out, bout2, f1s, ys2)


# speedup vs baseline: 1.0800x; 1.0645x over previous
import jax
import jax.numpy as jnp
from jax import lax
from jax.experimental import pallas as pl
from jax.experimental.pallas import tpu as pltpu
from jax.experimental.pallas import tpu_sc as plsc

N = 10000
E = 160000
D = 128
HID = 128
NH = 4
HC = 32
NC = 8
F_M = NH * HC * NC
FT = F_M + HID
CW = 128
NCH = FT // CW

NSC = 2
NTILE = 16
EPT = E // NTILE
EB = 125
NB = EPT // EB
NP = 10240
RPT = NP // NTILE

BN = 400
GRID = N // BN

_HIGH = lax.Precision.HIGHEST


def _sel_matrices():
    cc = lax.broadcasted_iota(jnp.int32, (HID, F_M), 1)
    rr = lax.broadcasted_iota(jnp.int32, (HID, F_M), 0)
    sk = (rr == (cc // (HC * NC)) * HC + (cc % (HC * NC)) // NC)
    return sk.astype(jnp.float32)


def _elu1(a):
    return 1.0 + jnp.where(a > 0, a, jnp.exp(jnp.minimum(a, 0.0)) - 1.0)


def _prep_body(x_ref, win_ref, bin_ref, wq_ref, bq_ref, wk_ref, bk_ref,
               wv_ref, bv_ref, q_ref, v_ref, *m0_refs):
    xb = x_ref[...]
    h = jnp.maximum(jnp.dot(xb, win_ref[...], precision=_HIGH) + bin_ref[...], 0.0)
    q = _elu1(jnp.dot(h, wq_ref[...], precision=_HIGH) + bq_ref[...])
    km = _elu1(jnp.dot(h, wk_ref[...], precision=_HIGH) + bk_ref[...])
    v = jnp.dot(h, wv_ref[...], precision=_HIGH) + bv_ref[...]
    q_ref[...] = q
    v_ref[...] = v
    sk = _sel_matrices()
    cc2 = lax.broadcasted_iota(jnp.int32, (NH * NC, F_M), 1)
    rr2 = lax.broadcasted_iota(jnp.int32, (NH * NC, F_M), 0)
    sv = (rr2 == (cc2 // (HC * NC)) * NC + cc2 % NC).astype(jnp.float32)
    m0 = jnp.dot(km, sk, precision=_HIGH) * jnp.dot(v, sv, precision=_HIGH)
    feat = jnp.concatenate([m0, km], axis=1)
    for c in range(NCH):
        m0_refs[c][...] = feat[:, c * CW:(c + 1) * CW]


def _tc_prep(x, w_in, b_in, wq, bq, wk, bk, wv, bv):
    node_spec = lambda w: pl.BlockSpec((BN, w), lambda i: (i, 0))
    full_spec = lambda a: pl.BlockSpec(a.shape, lambda i: (0,) * a.ndim)
    out_shape = ([jax.ShapeDtypeStruct((N, HID), jnp.float32),
                  jax.ShapeDtypeStruct((N, NH * NC), jnp.float32)] +
                 [jax.ShapeDtypeStruct((N, CW), jnp.float32) for _ in range(NCH)])
    out_specs = ([node_spec(HID), node_spec(NH * NC)] +
                 [node_spec(CW) for _ in range(NCH)])
    ws = (w_in, b_in, wq, bq, wk, bk, wv, bv)
    return pl.pallas_call(
        _prep_body,
        grid=(GRID,),
        in_specs=[node_spec(D)] + [full_spec(a) for a in ws],
        out_specs=out_specs,
        out_shape=out_shape,
    )(x, *ws)


def _sc_segsum(row2d, col2d, xs):
    nx = len(xs)
    mesh = plsc.VectorSubcoreMesh(core_axis_name="c", subcore_axis_name="s",
                                  num_cores=NSC, num_subcores=NTILE)
    out_type = [jax.ShapeDtypeStruct((NP, CW), jnp.float32) for _ in range(nx + 1)]
    scratch = [
        pltpu.VMEM((2, 8, EB), jnp.int32),
        pltpu.VMEM((2, 8, EB), jnp.int32),
        pltpu.VMEM((EB, CW), jnp.float32),
        pltpu.VMEM((EB, CW), jnp.float32),
        pltpu.VMEM((96, CW), jnp.float32),
        pltpu.VMEM_SHARED((NP, CW), jnp.float32),
        pltpu.SemaphoreType.DMA,
        pltpu.SemaphoreType.DMA,
        pltpu.SemaphoreType.DMA,
        pltpu.SemaphoreType.DMA,
        pltpu.SemaphoreType.DMA,
        pltpu.SemaphoreType.DMA,
    ]

    def body(row_ref, col_ref, *rest):
        x_refs = rest[:nx]
        y_refs = rest[nx:2 * nx + 1]
        (ridx2, cidx2, rows0, rows1, zbuf, acc,
         g0, g1, s0, s1, i0, i1) = rest[2 * nx + 1:]
        rows = (rows0, rows1)
        gsem = (g0, g1)
        ssem = (s0, s1)
        isem = (i0, i1)
        cid = lax.axis_index("c")
        sid = lax.axis_index("s")

        def zloop(i, carry):
            for j in range(CW // 16):
                zbuf[i, pl.ds(j * 16, 16)] = jnp.zeros((16,), jnp.float32)
            return carry
        lax.fori_loop(0, 96, zloop, 0)

        def process(x_ref, y_ref, ngroups, goff):
            for z in range(6):
                pltpu.sync_copy(zbuf, acc.at[pl.ds(sid * RPT + z * 96, 96)])
            pltpu.sync_copy(zbuf.at[pl.ds(0, 64)],
                            acc.at[pl.ds(sid * RPT + 576, 64)])
            plsc.subcore_barrier()

            def load_idx(grp, buf):
                base = pl.multiple_of(sid * NB + goff + grp * 8, 8)
                pltpu.async_copy(row_ref.at[pl.ds(base, 8)],
                                 ridx2.at[buf], isem[buf])
                pltpu.async_copy(col_ref.at[pl.ds(base, 8)],
                                 cidx2.at[buf], isem[buf])

            def wait_idx(grp, buf):
                base = pl.multiple_of(sid * NB + goff + grp * 8, 8)
                pltpu.make_async_copy(row_ref.at[pl.ds(base, 8)],
                                      ridx2.at[buf], isem[buf]).wait()
                pltpu.make_async_copy(col_ref.at[pl.ds(base, 8)],
                                      cidx2.at[buf], isem[buf]).wait()

            load_idx(0, 0)

            def oloop(bb, carry):
                for par in (0, 1):
                    @pl.when(bb % 2 == par)
                    def _(par=par):
                        @pl.when(bb + 1 < ngroups)
                        def _():
                            load_idx(bb + 1, 1 - par)
                        wait_idx(bb, par)
                        ridx = ridx2.at[par]
                        cidx = cidx2.at[par]
                        pltpu.async_copy(x_ref.at[ridx.at[0]], rows[0], gsem[0])
                        for k in range(8):
                            cur, nxt = k % 2, (k + 1) % 2
                            pltpu.make_async_copy(
                                x_ref.at[ridx.at[k]], rows[cur], gsem[cur]).wait()
                            if k >= 1:
                                pltpu.make_async_copy(
                                    rows[nxt], acc.at[cidx.at[k - 1]],
                                    ssem[nxt]).wait()
                            if k < 7:
                                pltpu.async_copy(
                                    x_ref.at[ridx.at[k + 1]], rows[nxt], gsem[nxt])
                            pltpu.async_copy(rows[cur], acc.at[cidx.at[k]],
                                             ssem[cur], add=True)
                        pltpu.make_async_copy(rows[1], acc.at[cidx.at[7]],
                                              ssem[1]).wait()
                return carry
            lax.fori_loop(0, ngroups, oloop, 0)
            plsc.subcore_barrier()
            pltpu.sync_copy(acc.at[pl.ds(sid * RPT, RPT)],
                            y_ref.at[pl.ds(sid * RPT, RPT)])

        for c in range(nx - 1):
            @pl.when(cid == (c % NSC))
            def _(c=c):
                process(x_refs[c], y_refs[c], NB // 8, 0)

        @pl.when(cid == 0)
        def _():
            process(x_refs[nx - 1], y_refs[nx - 1], NB // 16, 0)

        @pl.when(cid == 1)
        def _():
            process(x_refs[nx - 1], y_refs[nx], NB // 16, NB // 2)

    fn = pl.kernel(body, out_type=out_type, mesh=mesh, scratch_types=scratch)
    return fn(row2d, col2d, *xs)


def _add_body(a_ref, b_ref, o_ref):
    o_ref[...] = a_ref[...] + b_ref[...]


def _tc_add(a, b):
    spec = pl.BlockSpec((NP // 8, CW), lambda i: (i, 0))
    return pl.pallas_call(
        _add_body, grid=(8,), in_specs=[spec, spec], out_specs=spec,
        out_shape=jax.ShapeDtypeStruct((NP, CW), jnp.float32))(a, b)


def _final_body(q_ref, v_ref, hwe_ref, wout_ref, bout_ref, *rest):
    f1_refs = rest[:NCH]
    f2_refs = rest[NCH:2 * NCH + 1]
    out_ref = rest[2 * NCH + 1]
    qb = q_ref[...]
    sk = _sel_matrices()
    qexp = jnp.dot(qb, sk, precision=_HIGH)
    ccH = lax.broadcasted_iota(jnp.int32, (F_M, NH * NC), 0)
    ocH = lax.broadcasted_iota(jnp.int32, (F_M, NH * NC), 1)
    rh = (ocH == (ccH // (HC * NC)) * NC + ccH % NC).astype(jnp.float32)
    rrC = lax.broadcasted_iota(jnp.int32, (HID, NH * NC), 0)
    ocC = lax.broadcasted_iota(jnp.int32, (HID, NH * NC), 1)
    rc = (ocC // NC == rrC // HC).astype(jnp.float32)

    f1 = jnp.concatenate([r[...] for r in f1_refs], axis=1)
    f2 = jnp.concatenate([r[...] for r in f2_refs[:NCH - 1]] +
                         [f2_refs[NCH - 1][...] + f2_refs[NCH][...]], axis=1)
    m1, km1 = f1[:, :F_M], f1[:, F_M:]
    m2, km2 = f2[:, :F_M], f2[:, F_M:]
    h1 = jnp.dot(qexp * m1, rh, precision=_HIGH)
    c1 = jnp.dot(qb * km1, rc, precision=_HIGH)
    h2 = jnp.dot(qexp * m2, rh, precision=_HIGH)
    c2 = jnp.dot(qb * km2, rc, precision=_HIGH)
    hw = hwe_ref[...]
    hid = (v_ref[...] * hw[0:1, :] + hw[1:2, :] * h1 / (c1 + 1e-5)
           + hw[2:3, :] * h2 / (c2 + 1e-5))
    out_ref[...] = jnp.dot(hid, wout_ref[...], precision=_HIGH) + bout_ref[...]


def _tc_final(q, v, hwe, w_out, b_out, f1s, f2s):
    node_spec = lambda w: pl.BlockSpec((BN, w), lambda i: (i, 0))
    full_spec = lambda a: pl.BlockSpec(a.shape, lambda i: (0,) * a.ndim)
    ins = [q, v]
    in_specs = [node_spec(HID), node_spec(NH * NC)]
    ins += [hwe, w_out, b_out]
    in_specs += [full_spec(hwe), full_spec(w_out), full_spec(b_out)]
    ins += list(f1s) + list(f2s)
    in_specs += [node_spec(CW)] * (len(f1s) + len(f2s))
    return pl.pallas_call(
        _final_body,
        grid=(GRID,),
        in_specs=in_specs,
        out_specs=node_spec(NC),
        out_shape=jax.ShapeDtypeStruct((N, NC), jnp.float32),
    )(*ins)


def kernel(x, edge_index, W_in, b_in, WQ, bQ, WK, bK, WV, bV, W_out, b_out, hopwise):
    row2d = edge_index[0].reshape(NTILE * NB, EB)
    col2d = edge_index[1].reshape(NTILE * NB, EB)
    b_in2 = b_in.reshape(1, HID)
    bq2 = bQ.reshape(1, HID)
    bk2 = bK.reshape(1, HID)
    bv2 = bV.reshape(1, NH * NC)
    bout2 = b_out.reshape(1, NC)
    hwe = jnp.repeat(hopwise.T, NC, axis=1)

    q, v, *f0s = _tc_prep(x, W_in, b_in2, WQ, bq2, WK, bk2, WV, bv2)
    ys1 = _sc_segsum(row2d, col2d, list(f0s))
    km1 = _tc_add(ys1[NCH - 1], ys1[NCH])
    f1s = ys1[:NCH - 1] + [km1]
    ys2 = _sc_segsum(row2d, col2d, f1s)
    return _tc_final(q, v, hwe, W_out, bout2, f1s, ys2)


# speedup vs baseline: 1.0835x; 1.0032x over previous
import jax
import jax.numpy as jnp
from jax import lax
from jax.experimental import pallas as pl
from jax.experimental.pallas import tpu as pltpu
from jax.experimental.pallas import tpu_sc as plsc

N = 10000
E = 160000
D = 128
HID = 128
NH = 4
HC = 32
NC = 8
F_M = NH * HC * NC
FT = F_M + HID
CW = 128
NCH = FT // CW

NSC = 2
NTILE = 16
EPT = E // NTILE
EB = 125
NB = EPT // EB
NP = 10240
RPT = NP // NTILE

BN = 400
GRID = N // BN

_HIGH = lax.Precision.HIGHEST


def _sel_matrices():
    cc = lax.broadcasted_iota(jnp.int32, (HID, F_M), 1)
    rr = lax.broadcasted_iota(jnp.int32, (HID, F_M), 0)
    sk = (rr == (cc // (HC * NC)) * HC + (cc % (HC * NC)) // NC)
    return sk.astype(jnp.float32)


def _elu1(a):
    return 1.0 + jnp.where(a > 0, a, jnp.exp(jnp.minimum(a, 0.0)) - 1.0)


def _prep_body(x_ref, win_ref, bin_ref, wq_ref, bq_ref, wk_ref, bk_ref,
               wv_ref, bv_ref, q_ref, v_ref, *m0_refs):
    xb = x_ref[...]
    h = jnp.maximum(jnp.dot(xb, win_ref[...], precision=_HIGH) + bin_ref[...], 0.0)
    q = _elu1(jnp.dot(h, wq_ref[...], precision=_HIGH) + bq_ref[...])
    km = _elu1(jnp.dot(h, wk_ref[...], precision=_HIGH) + bk_ref[...])
    v = jnp.dot(h, wv_ref[...], precision=_HIGH) + bv_ref[...]
    q_ref[...] = q
    v_ref[...] = v
    sk = _sel_matrices()
    cc2 = lax.broadcasted_iota(jnp.int32, (NH * NC, F_M), 1)
    rr2 = lax.broadcasted_iota(jnp.int32, (NH * NC, F_M), 0)
    sv = (rr2 == (cc2 // (HC * NC)) * NC + cc2 % NC).astype(jnp.float32)
    m0 = jnp.dot(km, sk, precision=_HIGH) * jnp.dot(v, sv, precision=_HIGH)
    feat = jnp.concatenate([m0, km], axis=1)
    for c in range(NCH):
        m0_refs[c][...] = feat[:, c * CW:(c + 1) * CW]


def _tc_prep(x, w_in, b_in, wq, bq, wk, bk, wv, bv):
    node_spec = lambda w: pl.BlockSpec((BN, w), lambda i: (i, 0))
    full_spec = lambda a: pl.BlockSpec(a.shape, lambda i: (0,) * a.ndim)
    out_shape = ([jax.ShapeDtypeStruct((N, HID), jnp.float32),
                  jax.ShapeDtypeStruct((N, NH * NC), jnp.float32)] +
                 [jax.ShapeDtypeStruct((N, CW), jnp.float32) for _ in range(NCH)])
    out_specs = ([node_spec(HID), node_spec(NH * NC)] +
                 [node_spec(CW) for _ in range(NCH)])
    ws = (w_in, b_in, wq, bq, wk, bk, wv, bv)
    return pl.pallas_call(
        _prep_body,
        grid=(GRID,),
        in_specs=[node_spec(D)] + [full_spec(a) for a in ws],
        out_specs=out_specs,
        out_shape=out_shape,
    )(x, *ws)


def _sc_segsum(row2d, col2d, xs):
    nx = len(xs)
    mesh = plsc.VectorSubcoreMesh(core_axis_name="c", subcore_axis_name="s",
                                  num_cores=NSC, num_subcores=NTILE)
    out_type = [jax.ShapeDtypeStruct((NP, CW), jnp.float32) for _ in range(nx + 1)]
    scratch = [
        pltpu.VMEM((2, 8, EB), jnp.int32),
        pltpu.VMEM((2, 8, EB), jnp.int32),
        pltpu.VMEM((EB, CW), jnp.float32),
        pltpu.VMEM((EB, CW), jnp.float32),
        pltpu.VMEM((96, CW), jnp.float32),
        pltpu.VMEM_SHARED((NP, CW), jnp.float32),
        pltpu.SemaphoreType.DMA,
        pltpu.SemaphoreType.DMA,
        pltpu.SemaphoreType.DMA,
        pltpu.SemaphoreType.DMA,
        pltpu.SemaphoreType.DMA,
        pltpu.SemaphoreType.DMA,
    ]

    def body(row_ref, col_ref, *rest):
        x_refs = rest[:nx]
        y_refs = rest[nx:2 * nx + 1]
        (ridx2, cidx2, rows0, rows1, zbuf, acc,
         g0, g1, s0, s1, i0, i1) = rest[2 * nx + 1:]
        rows = (rows0, rows1)
        gsem = (g0, g1)
        ssem = (s0, s1)
        isem = (i0, i1)
        cid = lax.axis_index("c")
        sid = lax.axis_index("s")

        def zloop(i, carry):
            for j in range(CW // 16):
                zbuf[i, pl.ds(j * 16, 16)] = jnp.zeros((16,), jnp.float32)
            return carry
        lax.fori_loop(0, 96, zloop, 0)

        def process(x_ref, y_ref, ngroups, goff):
            for z in range(6):
                pltpu.sync_copy(zbuf, acc.at[pl.ds(sid * RPT + z * 96, 96)])
            pltpu.sync_copy(zbuf.at[pl.ds(0, 64)],
                            acc.at[pl.ds(sid * RPT + 576, 64)])
            plsc.subcore_barrier()

            def load_idx(grp, buf):
                base = pl.multiple_of(sid * NB + goff + grp * 8, 8)
                pltpu.async_copy(row_ref.at[pl.ds(base, 8)],
                                 ridx2.at[buf], isem[buf])
                pltpu.async_copy(col_ref.at[pl.ds(base, 8)],
                                 cidx2.at[buf], isem[buf])

            def wait_idx(grp, buf):
                base = pl.multiple_of(sid * NB + goff + grp * 8, 8)
                pltpu.make_async_copy(row_ref.at[pl.ds(base, 8)],
                                      ridx2.at[buf], isem[buf]).wait()
                pltpu.make_async_copy(col_ref.at[pl.ds(base, 8)],
                                      cidx2.at[buf], isem[buf]).wait()

            load_idx(0, 0)
            wait_idx(0, 0)
            pltpu.async_copy(x_ref.at[ridx2.at[0].at[0]], rows[0], gsem[0])

            def oloop(bb, carry):
                for par in (0, 1):
                    @pl.when(bb % 2 == par)
                    def _(par=par):
                        @pl.when(bb + 1 < ngroups)
                        def _():
                            load_idx(bb + 1, 1 - par)
                        ridx = ridx2.at[par]
                        cidx = cidx2.at[par]
                        for k in range(8):
                            cur, nxt = k % 2, (k + 1) % 2
                            pltpu.make_async_copy(
                                x_ref.at[ridx.at[k]], rows[cur], gsem[cur]).wait()
                            if k >= 1:
                                pltpu.make_async_copy(
                                    rows[nxt], acc.at[cidx.at[k - 1]],
                                    ssem[nxt]).wait()
                            if k < 7:
                                pltpu.async_copy(
                                    x_ref.at[ridx.at[k + 1]], rows[nxt], gsem[nxt])
                            pltpu.async_copy(rows[cur], acc.at[cidx.at[k]],
                                             ssem[cur], add=True)
                        pltpu.make_async_copy(rows[1], acc.at[cidx.at[7]],
                                              ssem[1]).wait()

                        @pl.when(bb + 1 < ngroups)
                        def _():
                            wait_idx(bb + 1, 1 - par)
                            pltpu.async_copy(
                                x_ref.at[ridx2.at[1 - par].at[0]],
                                rows[0], gsem[0])
                return carry
            lax.fori_loop(0, ngroups, oloop, 0)
            plsc.subcore_barrier()
            pltpu.sync_copy(acc.at[pl.ds(sid * RPT, RPT)],
                            y_ref.at[pl.ds(sid * RPT, RPT)])

        for c in range(nx - 1):
            @pl.when(cid == (c % NSC))
            def _(c=c):
                process(x_refs[c], y_refs[c], NB // 8, 0)

        @pl.when(cid == 0)
        def _():
            process(x_refs[nx - 1], y_refs[nx - 1], NB // 16, 0)

        @pl.when(cid == 1)
        def _():
            process(x_refs[nx - 1], y_refs[nx], NB // 16, NB // 2)

    fn = pl.kernel(body, out_type=out_type, mesh=mesh, scratch_types=scratch)
    return fn(row2d, col2d, *xs)


def _add_body(a_ref, b_ref, o_ref):
    o_ref[...] = a_ref[...] + b_ref[...]


def _tc_add(a, b):
    spec = pl.BlockSpec((NP // 8, CW), lambda i: (i, 0))
    return pl.pallas_call(
        _add_body, grid=(8,), in_specs=[spec, spec], out_specs=spec,
        out_shape=jax.ShapeDtypeStruct((NP, CW), jnp.float32))(a, b)


def _final_body(q_ref, v_ref, hwe_ref, wout_ref, bout_ref, *rest):
    f1_refs = rest[:NCH]
    f2_refs = rest[NCH:2 * NCH + 1]
    out_ref = rest[2 * NCH + 1]
    qb = q_ref[...]
    sk = _sel_matrices()
    qexp = jnp.dot(qb, sk, precision=_HIGH)
    ccH = lax.broadcasted_iota(jnp.int32, (F_M, NH * NC), 0)
    ocH = lax.broadcasted_iota(jnp.int32, (F_M, NH * NC), 1)
    rh = (ocH == (ccH // (HC * NC)) * NC + ccH % NC).astype(jnp.float32)
    rrC = lax.broadcasted_iota(jnp.int32, (HID, NH * NC), 0)
    ocC = lax.broadcasted_iota(jnp.int32, (HID, NH * NC), 1)
    rc = (ocC // NC == rrC // HC).astype(jnp.float32)

    f1 = jnp.concatenate([r[...] for r in f1_refs], axis=1)
    f2 = jnp.concatenate([r[...] for r in f2_refs[:NCH - 1]] +
                         [f2_refs[NCH - 1][...] + f2_refs[NCH][...]], axis=1)
    m1, km1 = f1[:, :F_M], f1[:, F_M:]
    m2, km2 = f2[:, :F_M], f2[:, F_M:]
    h1 = jnp.dot(qexp * m1, rh, precision=_HIGH)
    c1 = jnp.dot(qb * km1, rc, precision=_HIGH)
    h2 = jnp.dot(qexp * m2, rh, precision=_HIGH)
    c2 = jnp.dot(qb * km2, rc, precision=_HIGH)
    hw = hwe_ref[...]
    hid = (v_ref[...] * hw[0:1, :] + hw[1:2, :] * h1 / (c1 + 1e-5)
           + hw[2:3, :] * h2 / (c2 + 1e-5))
    out_ref[...] = jnp.dot(hid, wout_ref[...], precision=_HIGH) + bout_ref[...]


def _tc_final(q, v, hwe, w_out, b_out, f1s, f2s):
    node_spec = lambda w: pl.BlockSpec((BN, w), lambda i: (i, 0))
    full_spec = lambda a: pl.BlockSpec(a.shape, lambda i: (0,) * a.ndim)
    ins = [q, v]
    in_specs = [node_spec(HID), node_spec(NH * NC)]
    ins += [hwe, w_out, b_out]
    in_specs += [full_spec(hwe), full_spec(w_out), full_spec(b_out)]
    ins += list(f1s) + list(f2s)
    in_specs += [node_spec(CW)] * (len(f1s) + len(f2s))
    return pl.pallas_call(
        _final_body,
        grid=(GRID,),
        in_specs=in_specs,
        out_specs=node_spec(NC),
        out_shape=jax.ShapeDtypeStruct((N, NC), jnp.float32),
    )(*ins)


def kernel(x, edge_index, W_in, b_in, WQ, bQ, WK, bK, WV, bV, W_out, b_out, hopwise):
    row2d = edge_index[0].reshape(NTILE * NB, EB)
    col2d = edge_index[1].reshape(NTILE * NB, EB)
    b_in2 = b_in.reshape(1, HID)
    bq2 = bQ.reshape(1, HID)
    bk2 = bK.reshape(1, HID)
    bv2 = bV.reshape(1, NH * NC)
    bout2 = b_out.reshape(1, NC)
    hwe = jnp.repeat(hopwise.T, NC, axis=1)

    q, v, *f0s = _tc_prep(x, W_in, b_in2, WQ, bq2, WK, bk2, WV, bv2)
    ys1 = _sc_segsum(row2d, col2d, list(f0s))
    km1 = _tc_add(ys1[NCH - 1], ys1[NCH])
    f1s = ys1[:NCH - 1] + [km1]
    ys2 = _sc_segsum(row2d, col2d, f1s)
    return _tc_final(q, v, hwe, W_out, bout2, f1s, ys2)
